# Initial kernel scaffold; baseline (speedup 1.0000x reference)
import jax
import jax.numpy as jnp
from jax.experimental import pallas as pl


def kernel(x, edge_index, W1, b1, W2, b2):
    raise NotImplementedError("write your pallas kernel here")


# speedup vs baseline: 6.3653x; 6.3653x over previous
import functools

import jax
import jax.numpy as jnp
from jax import lax
from jax.experimental import pallas as pl
from jax.experimental.pallas import tpu as pltpu
from jax.experimental.pallas import tpu_sc as plsc

N = 10000
E = 160000
D = 256
DH = 128
DQ = 64
NC = 2
NS = 16
CH = 128
K = 80
EPAD = NS * K * CH
EPW = EPAD // (NC * NS)
ROWS_ACC = 10240
TRASH = N
RPT = ROWS_ACC // NS
RB = 1000
GRID = N // RB

@functools.lru_cache(maxsize=1)
def _sc_kernels():
    mesh = plsc.VectorSubcoreMesh(
        core_axis_name="c", subcore_axis_name="s",
        num_cores=NC, num_subcores=NS)

    deg_kernel = functools.partial(
        pl.kernel,
        out_type=jax.ShapeDtypeStruct((NC * NS, ROWS_ACC), jnp.float32),
        mesh=mesh,
        scratch_types=[
            pltpu.VMEM((EPW,), jnp.int32),
            pltpu.VMEM((ROWS_ACC,), jnp.float32),
        ],
        compiler_params=pltpu.CompilerParams(
            needs_layout_passes=False, use_tc_tiling_on_sc=False),
    )(_deg_body)

    scatter_kernel = functools.partial(
        pl.kernel,
        out_type=jax.ShapeDtypeStruct((2 * NC, ROWS_ACC, DQ), jnp.float32),
        mesh=mesh,
        scratch_types=[
            pltpu.VMEM((K, CH), jnp.int32),
            pltpu.VMEM((K, CH), jnp.int32),
            pltpu.VMEM((K, CH), jnp.int32),
            pltpu.VMEM((CH, DQ), jnp.float32),
            pltpu.VMEM((CH, DQ), jnp.float32),
            pltpu.VMEM((CH, DQ), jnp.float32),
            pltpu.VMEM_SHARED((ROWS_ACC, DQ), jnp.float32),
            pltpu.SemaphoreType.DMA,
            pltpu.SemaphoreType.DMA,
        ],
        compiler_params=pltpu.CompilerParams(
            needs_layout_passes=False, use_tc_tiling_on_sc=False),
    )(_scatter_body)
    return deg_kernel, scatter_kernel


def _deg_body(dst_hbm, out_hbm, dstv, hist):
    cid = lax.axis_index("c")
    sid = lax.axis_index("s")
    wid = sid * NC + cid

    def zb(i, carry):
        hist[pl.ds(i * 16, 16)] = jnp.zeros((16,), jnp.float32)
        return carry

    lax.fori_loop(0, ROWS_ACC // 16, zb, 0)
    pltpu.sync_copy(dst_hbm.at[wid], dstv)
    ones = jnp.ones((16,), jnp.float32)

    def body(j, carry):
        idx = dstv[pl.ds(j * 16, 16)]
        plsc.addupdate_scatter(hist, [idx], ones)
        return carry

    lax.fori_loop(0, EPW // 16, body, 0)
    pltpu.sync_copy(hist, out_hbm.at[wid])


def _scatter_body(g_hbm, src_hbm, dst_hbm, out_hbm,
                  srcv, gidx, didx, rows0, rows1, zbuf, acc, sem0, sem1):
    cid = lax.axis_index("c")
    sid = lax.axis_index("s")

    def zb(i, carry):
        r = i // (DQ // 16)
        c = (i % (DQ // 16)) * 16
        zbuf[r, pl.ds(c, 16)] = jnp.zeros((16,), jnp.float32)
        return carry

    lax.fori_loop(0, CH * (DQ // 16), zb, 0)

    pltpu.sync_copy(src_hbm.at[sid], srcv)
    pltpu.sync_copy(dst_hbm.at[sid], didx)

    for qpass in range(2):
        q = cid * 2 + qpass
        for k in range(RPT // CH):
            pltpu.sync_copy(zbuf, acc.at[pl.ds(sid * RPT + k * CH, CH)])

        def tb(i, carry):
            r = i // (CH // 16)
            c = (i % (CH // 16)) * 16
            s = srcv[r, pl.ds(c, 16)]
            gidx[r, pl.ds(c, 16)] = s * 4 + q
            return carry

        lax.fori_loop(0, K * (CH // 16), tb, 0)
        plsc.subcore_barrier()

        def body(i, carry):
            j0 = 2 * i
            j1 = 2 * i + 1
            cp0 = pltpu.async_copy(g_hbm.at[gidx.at[j0]], rows0, sem0)
            cp1 = pltpu.async_copy(g_hbm.at[gidx.at[j1]], rows1, sem1)
            cp0.wait()
            pltpu.sync_copy(rows0, acc.at[didx.at[j0]], add=True)
            cp1.wait()
            pltpu.sync_copy(rows1, acc.at[didx.at[j1]], add=True)
            return carry

        lax.fori_loop(0, K // 2, body, 0)
        plsc.subcore_barrier()
        pltpu.sync_copy(acc.at[pl.ds(sid * RPT, RPT)],
                        out_hbm.at[q, pl.ds(sid * RPT, RPT)])


def _mm_scale_body(deg_ref, x_ref, w_ref, out_ref):
    deg = jnp.sum(deg_ref[...], axis=1) + 1.0
    dinv = lax.rsqrt(deg)
    h = jnp.dot(x_ref[...], w_ref[...], preferred_element_type=jnp.float32)
    out_ref[...] = h * dinv[:, None]


def _combine_mm_body(deg_ref, sp_ref, g_ref, b_ref, w_ref, out_ref):
    deg = jnp.sum(deg_ref[...], axis=1) + 1.0
    dinv = lax.rsqrt(deg)
    sp = sp_ref[...]
    s = jnp.concatenate([sp[0], sp[1], sp[2], sp[3]], axis=-1)
    o = (s + g_ref[...]) * dinv[:, None] + b_ref[...]
    h = jnp.dot(o, w_ref[...], preferred_element_type=jnp.float32)
    out_ref[...] = h * dinv[:, None]


def _combine_stats_body(deg_ref, sp_ref, g_ref, b_ref, o_ref, stats_ref):
    i = pl.program_id(0)
    deg = jnp.sum(deg_ref[...], axis=1) + 1.0
    dinv = lax.rsqrt(deg)
    sp = sp_ref[...]
    s = jnp.concatenate([sp[0], sp[1], sp[2], sp[3]], axis=-1)
    o = (s + g_ref[...]) * dinv[:, None] + b_ref[...]
    o_ref[...] = o
    blk = jnp.stack([jnp.sum(o, axis=0), jnp.sum(o * o, axis=0)])

    @pl.when(i == 0)
    def _():
        stats_ref[...] = blk

    @pl.when(i > 0)
    def _():
        stats_ref[...] = stats_ref[...] + blk


def _norm_body(o_ref, stats_ref, out_ref):
    st = stats_ref[...]
    mean = st[0]
    nf = jnp.float32(N)
    mu = mean / nf
    var = (st[1] - nf * mu * mu) / (nf - 1.0)
    rstd = lax.rsqrt(var)
    out_ref[...] = (o_ref[...] - mu[None, :]) * rstd[None, :]


def _deg_spec():
    return pl.BlockSpec((RB, NC * NS), lambda i: (i, 0))


def _row_spec():
    return pl.BlockSpec((RB, D), lambda i: (i, 0))


def _sp_spec():
    return pl.BlockSpec((2 * NC, RB, DQ), lambda i: (0, i, 0))


def _full_spec(shape):
    return pl.BlockSpec(shape, lambda i: tuple(0 for _ in shape))


def kernel(x, edge_index, W1, b1, W2, b2):
    src = edge_index[0].astype(jnp.int32)
    dst = edge_index[1].astype(jnp.int32)
    pad = EPAD - E
    srcp = jnp.concatenate([src, jnp.zeros((pad,), jnp.int32)])
    dstp = jnp.concatenate([dst, jnp.full((pad,), TRASH, jnp.int32)])
    dst_w = dstp.reshape(NC * NS, EPW)
    src3 = srcp.reshape(NS, K, CH)
    dst3 = dstp.reshape(NS, K, CH)
    b1r = b1.reshape(1, D)
    b2r = b2.reshape(1, D)

    deg_kernel, scatter_kernel = _sc_kernels()
    deg_parts = deg_kernel(dst_w).T

    g1 = pl.pallas_call(
        _mm_scale_body,
        grid=(GRID,),
        in_specs=[_deg_spec(), _row_spec(), _full_spec((D, D))],
        out_specs=_row_spec(),
        out_shape=jax.ShapeDtypeStruct((N, D), jnp.float32),
    )(deg_parts, x, W1)

    s1 = scatter_kernel(g1.reshape(4 * N, DQ), src3, dst3)

    g2 = pl.pallas_call(
        _combine_mm_body,
        grid=(GRID,),
        in_specs=[_deg_spec(), _sp_spec(), _row_spec(),
                  _full_spec((1, D)), _full_spec((D, D))],
        out_specs=_row_spec(),
        out_shape=jax.ShapeDtypeStruct((N, D), jnp.float32),
    )(deg_parts, s1, g1, b1r, W2)

    s2 = scatter_kernel(g2.reshape(4 * N, DQ), src3, dst3)

    o2, stats = pl.pallas_call(
        _combine_stats_body,
        grid=(GRID,),
        in_specs=[_deg_spec(), _sp_spec(), _row_spec(), _full_spec((1, D))],
        out_specs=[_row_spec(), _full_spec((2, D))],
        out_shape=[jax.ShapeDtypeStruct((N, D), jnp.float32),
                   jax.ShapeDtypeStruct((2, D), jnp.float32)],
    )(deg_parts, s2, g2, b2r)

    out = pl.pallas_call(
        _norm_body,
        grid=(GRID,),
        in_specs=[_row_spec(), _full_spec((2, D))],
        out_specs=_row_spec(),
        out_shape=jax.ShapeDtypeStruct((N, D), jnp.float32),
    )(o2, stats)
    return out


# speedup vs baseline: 7.3846x; 1.1601x over previous
import functools

import jax
import jax.numpy as jnp
from jax import lax
from jax.experimental import pallas as pl
from jax.experimental.pallas import tpu as pltpu
from jax.experimental.pallas import tpu_sc as plsc

N = 10000
E = 160000
D = 256
DH = 128
DQ = 64
NC = 2
NS = 16
CH = 128
K = 80
NB = 4
EPAD = NS * K * CH
EPW = EPAD // (NC * NS)
ROWS_ACC = 10240
TRASH = N
RPT = ROWS_ACC // NS
RB = 1000
GRID = N // RB

@functools.lru_cache(maxsize=1)
def _sc_kernels():
    mesh = plsc.VectorSubcoreMesh(
        core_axis_name="c", subcore_axis_name="s",
        num_cores=NC, num_subcores=NS)

    deg_kernel = functools.partial(
        pl.kernel,
        out_type=jax.ShapeDtypeStruct((NC * NS, ROWS_ACC), jnp.float32),
        mesh=mesh,
        scratch_types=[
            pltpu.VMEM((EPW,), jnp.int32),
            pltpu.VMEM((ROWS_ACC,), jnp.float32),
        ],
        compiler_params=pltpu.CompilerParams(
            needs_layout_passes=False, use_tc_tiling_on_sc=False),
    )(_deg_body)

    scatter_kernel = functools.partial(
        pl.kernel,
        out_type=jax.ShapeDtypeStruct((2 * NC, ROWS_ACC, DQ), jnp.float32),
        mesh=mesh,
        scratch_types=[
            pltpu.VMEM((K, CH), jnp.int32),
            pltpu.VMEM((K, CH), jnp.int32),
            pltpu.VMEM((K, CH), jnp.int32),
            *[pltpu.VMEM((CH, DQ), jnp.float32) for _ in range(NB)],
            pltpu.VMEM((CH, DQ), jnp.float32),
            pltpu.VMEM_SHARED((ROWS_ACC, DQ), jnp.float32),
            *[pltpu.SemaphoreType.DMA for _ in range(2 * NB)],
        ],
        compiler_params=pltpu.CompilerParams(
            needs_layout_passes=False, use_tc_tiling_on_sc=False),
    )(_scatter_body)
    return deg_kernel, scatter_kernel


def _deg_body(dst_hbm, out_hbm, dstv, hist):
    cid = lax.axis_index("c")
    sid = lax.axis_index("s")
    wid = sid * NC + cid

    def zb(i, carry):
        hist[pl.ds(i * 16, 16)] = jnp.zeros((16,), jnp.float32)
        return carry

    lax.fori_loop(0, ROWS_ACC // 16, zb, 0)
    pltpu.sync_copy(dst_hbm.at[wid], dstv)
    ones = jnp.ones((16,), jnp.float32)

    def body(j, carry):
        idx = dstv[pl.ds(j * 16, 16)]
        plsc.addupdate_scatter(hist, [idx], ones)
        return carry

    lax.fori_loop(0, EPW // 16, body, 0)
    pltpu.sync_copy(hist, out_hbm.at[wid])


def _scatter_body(g_hbm, src_hbm, dst_hbm, out_hbm,
                  srcv, gidx, didx, *rest):
    rows = rest[:NB]
    zbuf, acc = rest[NB], rest[NB + 1]
    gsem = rest[NB + 2:NB + 2 + NB]
    ssem = rest[NB + 2 + NB:]
    cid = lax.axis_index("c")
    sid = lax.axis_index("s")

    def zb(i, carry):
        r = i // (DQ // 16)
        c = (i % (DQ // 16)) * 16
        zbuf[r, pl.ds(c, 16)] = jnp.zeros((16,), jnp.float32)
        return carry

    lax.fori_loop(0, CH * (DQ // 16), zb, 0)

    pltpu.sync_copy(src_hbm.at[sid], srcv)
    pltpu.sync_copy(dst_hbm.at[sid], didx)

    for qpass in range(2):
        q = cid * 2 + qpass
        for k in range(RPT // CH):
            pltpu.sync_copy(zbuf, acc.at[pl.ds(sid * RPT + k * CH, CH)])

        def tb(i, carry):
            r = i // (CH // 16)
            c = (i % (CH // 16)) * 16
            s = srcv[r, pl.ds(c, 16)]
            gidx[r, pl.ds(c, 16)] = s * 4 + q
            return carry

        lax.fori_loop(0, K * (CH // 16), tb, 0)
        plsc.subcore_barrier()

        def _gather(j, b):
            pltpu.async_copy(g_hbm.at[gidx.at[j]], rows[b], gsem[b])

        def _scatter(j, b):
            pltpu.async_copy(rows[b], acc.at[didx.at[j]], ssem[b], add=True)

        def _gwait(b):
            pltpu.make_async_copy(g_hbm.at[gidx.at[0]], rows[b],
                                  gsem[b]).wait()

        def _swait(b):
            pltpu.make_async_copy(rows[b], acc.at[didx.at[0]],
                                  ssem[b]).wait()

        for b in range(NB):
            _gather(b, b)

        def grp(g, carry):
            base = g * NB
            for b in range(NB):
                _gwait(b)
                _scatter(base - NB + b, b)
                _swait(b)
                _gather(base + b, b)
            return carry

        lax.fori_loop(1, K // NB, grp, 0)
        for b in range(NB):
            _gwait(b)
            _scatter(K - NB + b, b)
            _swait(b)
        plsc.subcore_barrier()
        pltpu.sync_copy(acc.at[pl.ds(sid * RPT, RPT)],
                        out_hbm.at[q, pl.ds(sid * RPT, RPT)])


def _mm_scale_body(deg_ref, x_ref, w_ref, out_ref):
    deg = jnp.sum(deg_ref[...], axis=1) + 1.0
    dinv = lax.rsqrt(deg)
    h = jnp.dot(x_ref[...], w_ref[...], preferred_element_type=jnp.float32)
    out_ref[...] = h * dinv[:, None]


def _combine_mm_body(deg_ref, sp_ref, g_ref, b_ref, w_ref, out_ref):
    deg = jnp.sum(deg_ref[...], axis=1) + 1.0
    dinv = lax.rsqrt(deg)
    sp = sp_ref[...]
    s = jnp.concatenate([sp[0], sp[1], sp[2], sp[3]], axis=-1)
    o = (s + g_ref[...]) * dinv[:, None] + b_ref[...]
    h = jnp.dot(o, w_ref[...], preferred_element_type=jnp.float32)
    out_ref[...] = h * dinv[:, None]


def _combine_stats_body(deg_ref, sp_ref, g_ref, b_ref, o_ref, stats_ref):
    i = pl.program_id(0)
    deg = jnp.sum(deg_ref[...], axis=1) + 1.0
    dinv = lax.rsqrt(deg)
    sp = sp_ref[...]
    s = jnp.concatenate([sp[0], sp[1], sp[2], sp[3]], axis=-1)
    o = (s + g_ref[...]) * dinv[:, None] + b_ref[...]
    o_ref[...] = o
    blk = jnp.stack([jnp.sum(o, axis=0), jnp.sum(o * o, axis=0)])

    @pl.when(i == 0)
    def _():
        stats_ref[...] = blk

    @pl.when(i > 0)
    def _():
        stats_ref[...] = stats_ref[...] + blk


def _norm_body(o_ref, stats_ref, out_ref):
    st = stats_ref[...]
    mean = st[0]
    nf = jnp.float32(N)
    mu = mean / nf
    var = (st[1] - nf * mu * mu) / (nf - 1.0)
    rstd = lax.rsqrt(var)
    out_ref[...] = (o_ref[...] - mu[None, :]) * rstd[None, :]


def _deg_spec():
    return pl.BlockSpec((RB, NC * NS), lambda i: (i, 0))


def _row_spec():
    return pl.BlockSpec((RB, D), lambda i: (i, 0))


def _sp_spec():
    return pl.BlockSpec((2 * NC, RB, DQ), lambda i: (0, i, 0))


def _full_spec(shape):
    return pl.BlockSpec(shape, lambda i: tuple(0 for _ in shape))


def kernel(x, edge_index, W1, b1, W2, b2):
    src = edge_index[0].astype(jnp.int32)
    dst = edge_index[1].astype(jnp.int32)
    pad = EPAD - E
    srcp = jnp.concatenate([src, jnp.zeros((pad,), jnp.int32)])
    dstp = jnp.concatenate([dst, jnp.full((pad,), TRASH, jnp.int32)])
    dst_w = dstp.reshape(NC * NS, EPW)
    src3 = srcp.reshape(NS, K, CH)
    dst3 = dstp.reshape(NS, K, CH)
    b1r = b1.reshape(1, D)
    b2r = b2.reshape(1, D)

    deg_kernel, scatter_kernel = _sc_kernels()
    deg_parts = deg_kernel(dst_w).T

    g1 = pl.pallas_call(
        _mm_scale_body,
        grid=(GRID,),
        in_specs=[_deg_spec(), _row_spec(), _full_spec((D, D))],
        out_specs=_row_spec(),
        out_shape=jax.ShapeDtypeStruct((N, D), jnp.float32),
    )(deg_parts, x, W1)

    s1 = scatter_kernel(g1.reshape(4 * N, DQ), src3, dst3)

    g2 = pl.pallas_call(
        _combine_mm_body,
        grid=(GRID,),
        in_specs=[_deg_spec(), _sp_spec(), _row_spec(),
                  _full_spec((1, D)), _full_spec((D, D))],
        out_specs=_row_spec(),
        out_shape=jax.ShapeDtypeStruct((N, D), jnp.float32),
    )(deg_parts, s1, g1, b1r, W2)

    s2 = scatter_kernel(g2.reshape(4 * N, DQ), src3, dst3)

    o2, stats = pl.pallas_call(
        _combine_stats_body,
        grid=(GRID,),
        in_specs=[_deg_spec(), _sp_spec(), _row_spec(), _full_spec((1, D))],
        out_specs=[_row_spec(), _full_spec((2, D))],
        out_shape=[jax.ShapeDtypeStruct((N, D), jnp.float32),
                   jax.ShapeDtypeStruct((2, D), jnp.float32)],
    )(deg_parts, s2, g2, b2r)

    out = pl.pallas_call(
        _norm_body,
        grid=(GRID,),
        in_specs=[_row_spec(), _full_spec((2, D))],
        out_specs=_row_spec(),
        out_shape=jax.ShapeDtypeStruct((N, D), jnp.float32),
    )(o2, stats)
    return out
